# baseline (speedup 1.0000x reference)
import functools

import jax
import jax.numpy as jnp
from jax import lax
from jax.experimental import pallas as pl
from jax.experimental.pallas import tpu as pltpu
from jax.experimental.pallas import tpu_sc as plsc

M = 262144
D = 128
K = 4
NU = M * K
NW = 32
CHUNK = NU // NW
R = 128
C = 128
CELLS = R * C


def _sc_body(skey_hbm, sval_hbm, mark_hbm, val_hbm, key_v, val_v, mark_v, vplane_v, usem):
    wid = lax.axis_index("s") * 2 + lax.axis_index("c")
    base = wid * CHUNK
    upd_copy = pltpu.make_async_copy(sval_hbm.at[pl.ds(base, CHUNK)], val_v, usem)
    upd_copy.start()
    pltpu.sync_copy(skey_hbm.at[pl.ds(base, CHUNK)], key_v.at[pl.ds(0, CHUNK)])

    iota = lax.iota(jnp.int32, 16)

    @pl.when(wid < NW - 1)
    def _():
        pltpu.sync_copy(skey_hbm.at[pl.ds(base + CHUNK, 8)],
                        key_v.at[pl.ds(CHUNK, 8)])

    @pl.when(wid == NW - 1)
    def _():
        key_v[pl.ds(CHUNK, 16)] = jnp.full((16,), -2, jnp.int32)

    zeros_i = jnp.zeros((16,), jnp.int32)
    zeros_f = jnp.zeros((16,), jnp.float32)
    ones_i = jnp.full((16,), 1, jnp.int32)

    def init_body(i, _):
        mark_v[pl.ds(i * 16, 16)] = zeros_i
        vplane_v[pl.ds(i * 16, 16)] = zeros_f
        return 0

    lax.fori_loop(0, CELLS // 16, init_body, 0)
    upd_copy.wait()

    def scan_body(v, _):
        cur = key_v[pl.ds(v * 16, 16)]
        nxt = plsc.load_gather(key_v, [v * 16 + 1 + iota])
        keep = cur != nxt
        val = val_v[pl.ds(v * 16, 16)]
        plsc.store_scatter(mark_v, [cur], ones_i, mask=keep)
        plsc.store_scatter(vplane_v, [cur], val, mask=keep)
        return 0

    lax.fori_loop(0, CHUNK // 16, scan_body, 0)

    pltpu.sync_copy(mark_v, mark_hbm.at[wid])
    pltpu.sync_copy(vplane_v, val_hbm.at[wid])


ROWS_PER_TILE = M // NW


def _sc_copy_body(op_hbm, out_hbm):
    wid = lax.axis_index("s") * 2 + lax.axis_index("c")
    r0 = wid * ROWS_PER_TILE
    pltpu.sync_copy(op_hbm.at[pl.ds(r0, ROWS_PER_TILE), :],
                    out_hbm.at[pl.ds(r0, ROWS_PER_TILE), :])


@functools.cache
def _sc_copy():
    return pl.kernel(
        _sc_copy_body,
        mesh=plsc.VectorSubcoreMesh(core_axis_name="c", subcore_axis_name="s"),
        out_type=jax.ShapeDtypeStruct((M, D), jnp.float32),
        compiler_params=pltpu.CompilerParams(needs_layout_passes=False),
    )


@functools.cache
def _sc_scatter():
    return pl.kernel(
        _sc_body,
        mesh=plsc.VectorSubcoreMesh(core_axis_name="c", subcore_axis_name="s"),
        out_type=[
            jax.ShapeDtypeStruct((NW, CELLS), jnp.int32),
            jax.ShapeDtypeStruct((NW, CELLS), jnp.float32),
        ],
        scratch_types=[
            pltpu.VMEM((CHUNK + 16,), jnp.int32),
            pltpu.VMEM((CHUNK,), jnp.float32),
            pltpu.VMEM((CELLS,), jnp.int32),
            pltpu.VMEM((CELLS,), jnp.float32),
            pltpu.SemaphoreType.DMA,
        ],
        compiler_params=pltpu.CompilerParams(needs_layout_passes=False),
    )


def _merge_body(buf_ref, mark_ref, val_ref, out_ref):
    patch = buf_ref[...]
    for t in range(NW):
        patch = jnp.where(mark_ref[t] != 0, val_ref[t], patch)
    out_ref[...] = patch


def kernel(operand, scatter_indices, updates):
    si = scatter_indices.astype(jnp.int32)
    keys = (si[..., 0] * C + si[..., 1]).reshape(-1)
    upd = updates.reshape(-1)

    out_buf = _sc_copy()(operand)

    skey, sval = lax.sort((keys, upd), dimension=0, is_stable=False, num_keys=1)
    mark, vals = _sc_scatter()(skey, sval)

    return pl.pallas_call(
        _merge_body,
        grid=(1,),
        in_specs=[
            pl.BlockSpec((R, C), lambda i: (0, 0)),
            pl.BlockSpec((NW, R, C), lambda i: (0, 0, 0)),
            pl.BlockSpec((NW, R, C), lambda i: (0, 0, 0)),
        ],
        out_specs=pl.BlockSpec((R, C), lambda i: (0, 0)),
        out_shape=jax.ShapeDtypeStruct((M, D), jnp.float32),
        input_output_aliases={0: 0},
    )(out_buf, mark.reshape(NW, R, C), vals.reshape(NW, R, C))


# speedup vs baseline: 4.2770x; 4.2770x over previous
import functools

import jax
import jax.numpy as jnp
from jax import lax
from jax.experimental import pallas as pl
from jax.experimental.pallas import tpu as pltpu
from jax.experimental.pallas import tpu_sc as plsc

M = 262144
D = 128
K = 4
NU = M * K
NW = 32
CHUNK = NU // NW
R = 128
C = 128
CELLS = R * C


def _sc_body(skey_hbm, sval_hbm, mark_hbm, val_hbm, key_v, val_v, mark_v, vplane_v, usem):
    wid = lax.axis_index("s") * 2 + lax.axis_index("c")
    base = wid * CHUNK
    upd_copy = pltpu.make_async_copy(sval_hbm.at[pl.ds(base, CHUNK)], val_v, usem)
    upd_copy.start()
    pltpu.sync_copy(skey_hbm.at[pl.ds(base, CHUNK)], key_v.at[pl.ds(0, CHUNK)])

    iota = lax.iota(jnp.int32, 16)

    @pl.when(wid < NW - 1)
    def _():
        pltpu.sync_copy(skey_hbm.at[pl.ds(base + CHUNK, 8)],
                        key_v.at[pl.ds(CHUNK, 8)])

    @pl.when(wid == NW - 1)
    def _():
        key_v[pl.ds(CHUNK, 16)] = jnp.full((16,), -2, jnp.int32)

    zeros_i = jnp.zeros((16,), jnp.int32)
    zeros_f = jnp.zeros((16,), jnp.float32)
    ones_i = jnp.full((16,), 1, jnp.int32)

    def init_body(i, _):
        mark_v[pl.ds(i * 16, 16)] = zeros_i
        vplane_v[pl.ds(i * 16, 16)] = zeros_f
        return 0

    lax.fori_loop(0, CELLS // 16, init_body, 0)
    upd_copy.wait()

    def scan_body(v, _):
        cur = key_v[pl.ds(v * 16, 16)]
        nxt = plsc.load_gather(key_v, [v * 16 + 1 + iota])
        keep = cur != nxt
        val = val_v[pl.ds(v * 16, 16)]
        plsc.store_scatter(mark_v, [cur], ones_i, mask=keep)
        plsc.store_scatter(vplane_v, [cur], val, mask=keep)
        return 0

    lax.fori_loop(0, CHUNK // 16, scan_body, 0)

    pltpu.sync_copy(mark_v, mark_hbm.at[wid])
    pltpu.sync_copy(vplane_v, val_hbm.at[wid])


ROWS_PER_TILE = M // NW
PIECE_ROWS = 256
NPIECE = ROWS_PER_TILE // PIECE_ROWS


def _sc_copy_body(op_hbm, out_hbm, buf0, buf1, rs0, rs1, ws0, ws1):
    wid = lax.axis_index("s") * 2 + lax.axis_index("c")
    r0 = wid * ROWS_PER_TILE
    bufs = (buf0, buf1)
    rsems = (rs0, rs1)
    wsems = (ws0, ws1)

    def rd(i):
        return pltpu.make_async_copy(
            op_hbm.at[pl.ds(r0 + i * PIECE_ROWS, PIECE_ROWS), :],
            bufs[i % 2], rsems[i % 2])

    def wr(i):
        return pltpu.make_async_copy(
            bufs[i % 2],
            out_hbm.at[pl.ds(r0 + i * PIECE_ROWS, PIECE_ROWS), :],
            wsems[i % 2])

    rd(0).start()
    for i in range(NPIECE):
        rd(i).wait()
        wr(i).start()
        if i + 1 < NPIECE:
            if i >= 1:
                wr(i - 1).wait()
            rd(i + 1).start()
    wr(NPIECE - 2).wait()
    wr(NPIECE - 1).wait()


@functools.cache
def _sc_copy():
    return pl.kernel(
        _sc_copy_body,
        mesh=plsc.VectorSubcoreMesh(core_axis_name="c", subcore_axis_name="s"),
        out_type=jax.ShapeDtypeStruct((M, D), jnp.float32),
        scratch_types=[
            pltpu.VMEM((PIECE_ROWS, D), jnp.float32),
            pltpu.VMEM((PIECE_ROWS, D), jnp.float32),
            pltpu.SemaphoreType.DMA,
            pltpu.SemaphoreType.DMA,
            pltpu.SemaphoreType.DMA,
            pltpu.SemaphoreType.DMA,
        ],
        compiler_params=pltpu.CompilerParams(needs_layout_passes=False),
    )


@functools.cache
def _sc_scatter():
    return pl.kernel(
        _sc_body,
        mesh=plsc.VectorSubcoreMesh(core_axis_name="c", subcore_axis_name="s"),
        out_type=[
            jax.ShapeDtypeStruct((NW, CELLS), jnp.int32),
            jax.ShapeDtypeStruct((NW, CELLS), jnp.float32),
        ],
        scratch_types=[
            pltpu.VMEM((CHUNK + 16,), jnp.int32),
            pltpu.VMEM((CHUNK,), jnp.float32),
            pltpu.VMEM((CELLS,), jnp.int32),
            pltpu.VMEM((CELLS,), jnp.float32),
            pltpu.SemaphoreType.DMA,
        ],
        compiler_params=pltpu.CompilerParams(needs_layout_passes=False),
    )


def _merge_body(buf_ref, mark_ref, val_ref, out_ref):
    patch = buf_ref[...]
    for t in range(NW):
        patch = jnp.where(mark_ref[t] != 0, val_ref[t], patch)
    out_ref[...] = patch


def kernel(operand, scatter_indices, updates):
    si = scatter_indices.astype(jnp.int32)
    keys = (si[..., 0] * C + si[..., 1]).reshape(-1)
    upd = updates.reshape(-1)

    out_buf = _sc_copy()(operand)

    skey, sval = lax.sort((keys, upd), dimension=0, is_stable=False, num_keys=1)
    mark, vals = _sc_scatter()(skey, sval)

    return pl.pallas_call(
        _merge_body,
        grid=(1,),
        in_specs=[
            pl.BlockSpec((R, C), lambda i: (0, 0)),
            pl.BlockSpec((NW, R, C), lambda i: (0, 0, 0)),
            pl.BlockSpec((NW, R, C), lambda i: (0, 0, 0)),
        ],
        out_specs=pl.BlockSpec((R, C), lambda i: (0, 0)),
        out_shape=jax.ShapeDtypeStruct((M, D), jnp.float32),
        input_output_aliases={0: 0},
    )(out_buf, mark.reshape(NW, R, C), vals.reshape(NW, R, C))


# speedup vs baseline: 5.6968x; 1.3320x over previous
import functools

import jax
import jax.numpy as jnp
from jax import lax
from jax.experimental import pallas as pl
from jax.experimental.pallas import tpu as pltpu
from jax.experimental.pallas import tpu_sc as plsc

M = 262144
D = 128
K = 4
NU = M * K
NW = 32
CHUNK = NU // NW
R = 128
C = 128
CELLS = R * C


def _sc_body(skey_hbm, sval_hbm, mark_hbm, val_hbm, key_v, val_v, mark_v, vplane_v, usem):
    wid = lax.axis_index("s") * 2 + lax.axis_index("c")
    base = wid * CHUNK
    upd_copy = pltpu.make_async_copy(sval_hbm.at[pl.ds(base, CHUNK)], val_v, usem)
    upd_copy.start()
    pltpu.sync_copy(skey_hbm.at[pl.ds(base, CHUNK)], key_v.at[pl.ds(0, CHUNK)])

    iota = lax.iota(jnp.int32, 16)

    @pl.when(wid < NW - 1)
    def _():
        pltpu.sync_copy(skey_hbm.at[pl.ds(base + CHUNK, 8)],
                        key_v.at[pl.ds(CHUNK, 8)])

    @pl.when(wid == NW - 1)
    def _():
        key_v[pl.ds(CHUNK, 16)] = jnp.full((16,), -2, jnp.int32)

    zeros_i = jnp.zeros((16,), jnp.int32)
    zeros_f = jnp.zeros((16,), jnp.float32)
    ones_i = jnp.full((16,), 1, jnp.int32)

    def init_body(i, _):
        mark_v[pl.ds(i * 16, 16)] = zeros_i
        vplane_v[pl.ds(i * 16, 16)] = zeros_f
        return 0

    lax.fori_loop(0, CELLS // 16, init_body, 0)
    upd_copy.wait()

    def scan_body(v, _):
        cur = key_v[pl.ds(v * 16, 16)]
        nxt = plsc.load_gather(key_v, [v * 16 + 1 + iota])
        keep = cur != nxt
        val = val_v[pl.ds(v * 16, 16)]
        plsc.store_scatter(mark_v, [cur], ones_i, mask=keep)
        plsc.store_scatter(vplane_v, [cur], val, mask=keep)
        return 0

    lax.fori_loop(0, CHUNK // 16, scan_body, 0)

    pltpu.sync_copy(mark_v, mark_hbm.at[wid])
    pltpu.sync_copy(vplane_v, val_hbm.at[wid])


ROWS_PER_TILE = M // NW
PIECE_ROWS = 256
NPIECE = ROWS_PER_TILE // PIECE_ROWS


def _sc_copy_body(op_hbm, out_hbm, buf0, buf1, rs0, rs1, ws0, ws1):
    wid = lax.axis_index("s") * 2 + lax.axis_index("c")
    r0 = wid * ROWS_PER_TILE
    bufs = (buf0, buf1)
    rsems = (rs0, rs1)
    wsems = (ws0, ws1)

    def rd(i):
        return pltpu.make_async_copy(
            op_hbm.at[pl.ds(r0 + i * PIECE_ROWS, PIECE_ROWS), :],
            bufs[i % 2], rsems[i % 2])

    def wr(i):
        return pltpu.make_async_copy(
            bufs[i % 2],
            out_hbm.at[pl.ds(r0 + i * PIECE_ROWS, PIECE_ROWS), :],
            wsems[i % 2])

    rd(0).start()
    for i in range(NPIECE):
        rd(i).wait()
        wr(i).start()
        if i + 1 < NPIECE:
            if i >= 1:
                wr(i - 1).wait()
            rd(i + 1).start()
    wr(NPIECE - 2).wait()
    wr(NPIECE - 1).wait()


@functools.cache
def _sc_copy():
    return pl.kernel(
        _sc_copy_body,
        mesh=plsc.VectorSubcoreMesh(core_axis_name="c", subcore_axis_name="s"),
        out_type=jax.ShapeDtypeStruct((M, D), jnp.float32),
        scratch_types=[
            pltpu.VMEM((PIECE_ROWS, D), jnp.float32),
            pltpu.VMEM((PIECE_ROWS, D), jnp.float32),
            pltpu.SemaphoreType.DMA,
            pltpu.SemaphoreType.DMA,
            pltpu.SemaphoreType.DMA,
            pltpu.SemaphoreType.DMA,
        ],
        compiler_params=pltpu.CompilerParams(needs_layout_passes=False),
        cost_estimate=pl.CostEstimate(
            flops=0, bytes_accessed=2 * M * D * 4, transcendentals=0),
    )


@functools.cache
def _sc_scatter():
    return pl.kernel(
        _sc_body,
        mesh=plsc.VectorSubcoreMesh(core_axis_name="c", subcore_axis_name="s"),
        out_type=[
            jax.ShapeDtypeStruct((NW, CELLS), jnp.int32),
            jax.ShapeDtypeStruct((NW, CELLS), jnp.float32),
        ],
        scratch_types=[
            pltpu.VMEM((CHUNK + 16,), jnp.int32),
            pltpu.VMEM((CHUNK,), jnp.float32),
            pltpu.VMEM((CELLS,), jnp.int32),
            pltpu.VMEM((CELLS,), jnp.float32),
            pltpu.SemaphoreType.DMA,
        ],
        compiler_params=pltpu.CompilerParams(needs_layout_passes=False),
    )


MR = M // NW
HALF = MR // 2


def _sc_prelude_body(si_hbm, upd_hbm, keys_hbm, updlin_hbm,
                     si_v, upd4_v, key_v, updlin_v):
    wid = lax.axis_index("s") * 2 + lax.axis_index("c")
    iota = lax.iota(jnp.int32, 16)
    zeros = jnp.zeros((16,), jnp.int32)
    ones = jnp.full((16,), 1, jnp.int32)

    for h in range(2):
        m0 = wid * MR + h * HALF
        pltpu.sync_copy(si_hbm.at[:, :, pl.ds(m0, HALF)], si_v)
        pltpu.sync_copy(upd_hbm.at[:, pl.ds(m0, HALF)], upd4_v)

        def body(v, _):
            jv = v * 16 + iota
            mrel = jv >> 2
            k = jv & 3
            i0 = plsc.load_gather(si_v, [k, zeros, mrel])
            i1 = plsc.load_gather(si_v, [k, ones, mrel])
            key_v[pl.ds(v * 16, 16)] = i0 * C + i1
            updlin_v[pl.ds(v * 16, 16)] = plsc.load_gather(upd4_v, [k, mrel])
            return 0

        lax.fori_loop(0, (HALF * K) // 16, body, 0)
        j0 = wid * (MR * K) + h * (HALF * K)
        pltpu.sync_copy(key_v, keys_hbm.at[pl.ds(j0, HALF * K)])
        pltpu.sync_copy(updlin_v, updlin_hbm.at[pl.ds(j0, HALF * K)])


@functools.cache
def _sc_prelude():
    return pl.kernel(
        _sc_prelude_body,
        mesh=plsc.VectorSubcoreMesh(core_axis_name="c", subcore_axis_name="s"),
        out_type=[
            jax.ShapeDtypeStruct((NU,), jnp.int32),
            jax.ShapeDtypeStruct((NU,), jnp.float32),
        ],
        scratch_types=[
            pltpu.VMEM((K, 2, HALF), jnp.int32),
            pltpu.VMEM((K, HALF), jnp.float32),
            pltpu.VMEM((HALF * K,), jnp.int32),
            pltpu.VMEM((HALF * K,), jnp.float32),
        ],
        compiler_params=pltpu.CompilerParams(needs_layout_passes=False),
    )


def _merge_body(buf_ref, mark_ref, val_ref, out_ref):
    patch = buf_ref[...]
    for t in range(NW):
        patch = jnp.where(mark_ref[t] != 0, val_ref[t], patch)
    out_ref[...] = patch


def kernel(operand, scatter_indices, updates):
    si_t = jnp.transpose(scatter_indices.astype(jnp.int32), (1, 2, 0))
    upd_t = jnp.transpose(updates, (1, 0))
    keys, upd = _sc_prelude()(si_t, upd_t)

    out_buf = _sc_copy()(operand)

    skey, sval = lax.sort((keys, upd), dimension=0, is_stable=False, num_keys=1)
    mark, vals = _sc_scatter()(skey, sval)

    return pl.pallas_call(
        _merge_body,
        grid=(1,),
        in_specs=[
            pl.BlockSpec((R, C), lambda i: (0, 0)),
            pl.BlockSpec((NW, R, C), lambda i: (0, 0, 0)),
            pl.BlockSpec((NW, R, C), lambda i: (0, 0, 0)),
        ],
        out_specs=pl.BlockSpec((R, C), lambda i: (0, 0)),
        out_shape=jax.ShapeDtypeStruct((M, D), jnp.float32),
        input_output_aliases={0: 0},
    )(out_buf, mark.reshape(NW, R, C), vals.reshape(NW, R, C))


# speedup vs baseline: 5.8857x; 1.0332x over previous
import functools

import jax
import jax.numpy as jnp
from jax import lax
from jax.experimental import pallas as pl
from jax.experimental.pallas import tpu as pltpu
from jax.experimental.pallas import tpu_sc as plsc

M = 262144
D = 128
K = 4
NU = M * K
NW = 32
CHUNK = NU // NW
R = 128
C = 128
CELLS = R * C


def _sc_body(skey_hbm, sval_hbm, mark_hbm, val_hbm, key_v, val_v, mark_v, vplane_v, usem):
    wid = lax.axis_index("s") * 2 + lax.axis_index("c")
    base = wid * CHUNK
    upd_copy = pltpu.make_async_copy(sval_hbm.at[pl.ds(base, CHUNK)], val_v, usem)
    upd_copy.start()
    pltpu.sync_copy(skey_hbm.at[pl.ds(base, CHUNK)], key_v.at[pl.ds(0, CHUNK)])

    iota = lax.iota(jnp.int32, 16)

    @pl.when(wid < NW - 1)
    def _():
        pltpu.sync_copy(skey_hbm.at[pl.ds(base + CHUNK, 8)],
                        key_v.at[pl.ds(CHUNK, 8)])

    @pl.when(wid == NW - 1)
    def _():
        key_v[pl.ds(CHUNK, 16)] = jnp.full((16,), -2, jnp.int32)

    zeros_i = jnp.zeros((16,), jnp.int32)
    zeros_f = jnp.zeros((16,), jnp.float32)
    ones_i = jnp.full((16,), 1, jnp.int32)

    def init_body(i, _):
        mark_v[pl.ds(i * 16, 16)] = zeros_i
        vplane_v[pl.ds(i * 16, 16)] = zeros_f
        return 0

    lax.fori_loop(0, CELLS // 16, init_body, 0)
    upd_copy.wait()

    def scan_body(v, _):
        cur = key_v[pl.ds(v * 16, 16)]
        nxt = plsc.load_gather(key_v, [v * 16 + 1 + iota])
        keep = cur != nxt
        val = val_v[pl.ds(v * 16, 16)]
        plsc.store_scatter(mark_v, [cur], ones_i, mask=keep)
        plsc.store_scatter(vplane_v, [cur], val, mask=keep)
        return 0

    lax.fori_loop(0, CHUNK // 16, scan_body, 0)

    pltpu.sync_copy(mark_v, mark_hbm.at[wid])
    pltpu.sync_copy(vplane_v, val_hbm.at[wid])


@functools.cache
def _sc_scatter():
    return pl.kernel(
        _sc_body,
        mesh=plsc.VectorSubcoreMesh(core_axis_name="c", subcore_axis_name="s"),
        out_type=[
            jax.ShapeDtypeStruct((NW, CELLS), jnp.int32),
            jax.ShapeDtypeStruct((NW, CELLS), jnp.float32),
        ],
        scratch_types=[
            pltpu.VMEM((CHUNK + 16,), jnp.int32),
            pltpu.VMEM((CHUNK,), jnp.float32),
            pltpu.VMEM((CELLS,), jnp.int32),
            pltpu.VMEM((CELLS,), jnp.float32),
            pltpu.SemaphoreType.DMA,
        ],
        compiler_params=pltpu.CompilerParams(needs_layout_passes=False),
    )


MR = M // NW
HALF = MR // 2


def _sc_prelude_body(si_hbm, upd_hbm, keys_hbm, updlin_hbm,
                     si_v, upd4_v, key_v, updlin_v):
    wid = lax.axis_index("s") * 2 + lax.axis_index("c")
    iota = lax.iota(jnp.int32, 16)
    zeros = jnp.zeros((16,), jnp.int32)
    ones = jnp.full((16,), 1, jnp.int32)

    for h in range(2):
        m0 = wid * MR + h * HALF
        pltpu.sync_copy(si_hbm.at[:, :, pl.ds(m0, HALF)], si_v)
        pltpu.sync_copy(upd_hbm.at[:, pl.ds(m0, HALF)], upd4_v)

        def body(v, _):
            jv = v * 16 + iota
            mrel = jv >> 2
            k = jv & 3
            i0 = plsc.load_gather(si_v, [k, zeros, mrel])
            i1 = plsc.load_gather(si_v, [k, ones, mrel])
            key_v[pl.ds(v * 16, 16)] = i0 * C + i1
            updlin_v[pl.ds(v * 16, 16)] = plsc.load_gather(upd4_v, [k, mrel])
            return 0

        lax.fori_loop(0, (HALF * K) // 16, body, 0)
        j0 = wid * (MR * K) + h * (HALF * K)
        pltpu.sync_copy(key_v, keys_hbm.at[pl.ds(j0, HALF * K)])
        pltpu.sync_copy(updlin_v, updlin_hbm.at[pl.ds(j0, HALF * K)])


@functools.cache
def _sc_prelude():
    return pl.kernel(
        _sc_prelude_body,
        mesh=plsc.VectorSubcoreMesh(core_axis_name="c", subcore_axis_name="s"),
        out_type=[
            jax.ShapeDtypeStruct((NU,), jnp.int32),
            jax.ShapeDtypeStruct((NU,), jnp.float32),
        ],
        scratch_types=[
            pltpu.VMEM((K, 2, HALF), jnp.int32),
            pltpu.VMEM((K, HALF), jnp.float32),
            pltpu.VMEM((HALF * K,), jnp.int32),
            pltpu.VMEM((HALF * K,), jnp.float32),
        ],
        compiler_params=pltpu.CompilerParams(needs_layout_passes=False),
    )


ROWS_PER_BLK = 4096


def _tc_copy_body(op_ref, out_ref):
    out_ref[...] = op_ref[...]


def _merge_body(buf_ref, mark_ref, val_ref, out_ref):
    patch = buf_ref[...]
    for t in range(NW):
        patch = jnp.where(mark_ref[t] != 0, val_ref[t], patch)
    out_ref[...] = patch


def kernel(operand, scatter_indices, updates):
    si_t = jnp.transpose(scatter_indices.astype(jnp.int32), (1, 2, 0))
    upd_t = jnp.transpose(updates, (1, 0))
    keys, upd = _sc_prelude()(si_t, upd_t)

    out_buf = pl.pallas_call(
        _tc_copy_body,
        grid=(M // ROWS_PER_BLK,),
        in_specs=[pl.BlockSpec((ROWS_PER_BLK, D), lambda i: (i, 0))],
        out_specs=pl.BlockSpec((ROWS_PER_BLK, D), lambda i: (i, 0)),
        out_shape=jax.ShapeDtypeStruct((M, D), jnp.float32),
    )(operand)

    skey, sval = lax.sort((keys, upd), dimension=0, is_stable=False, num_keys=1)
    mark, vals = _sc_scatter()(skey, sval)

    return pl.pallas_call(
        _merge_body,
        grid=(1,),
        in_specs=[
            pl.BlockSpec((R, C), lambda i: (0, 0)),
            pl.BlockSpec((NW, R, C), lambda i: (0, 0, 0)),
            pl.BlockSpec((NW, R, C), lambda i: (0, 0, 0)),
        ],
        out_specs=pl.BlockSpec((R, C), lambda i: (0, 0)),
        out_shape=jax.ShapeDtypeStruct((M, D), jnp.float32),
        input_output_aliases={0: 0},
    )(out_buf, mark.reshape(NW, R, C), vals.reshape(NW, R, C))


# speedup vs baseline: 5.8878x; 1.0004x over previous
import functools

import jax
import jax.numpy as jnp
from jax import lax
from jax.experimental import pallas as pl
from jax.experimental.pallas import tpu as pltpu
from jax.experimental.pallas import tpu_sc as plsc

M = 262144
D = 128
K = 4
NU = M * K
NW = 32
CHUNK = NU // NW
R = 128
C = 128
CELLS = R * C


def _sc_body(skey_hbm, sval_hbm, mark_hbm, val_hbm, key_v, val_v, mark_v, vplane_v, usem):
    wid = lax.axis_index("s") * 2 + lax.axis_index("c")
    base = wid * CHUNK
    upd_copy = pltpu.make_async_copy(sval_hbm.at[pl.ds(base, CHUNK)], val_v, usem)
    upd_copy.start()
    pltpu.sync_copy(skey_hbm.at[pl.ds(base, CHUNK)], key_v.at[pl.ds(0, CHUNK)])

    iota = lax.iota(jnp.int32, 16)

    @pl.when(wid < NW - 1)
    def _():
        pltpu.sync_copy(skey_hbm.at[pl.ds(base + CHUNK, 8)],
                        key_v.at[pl.ds(CHUNK, 8)])

    @pl.when(wid == NW - 1)
    def _():
        key_v[pl.ds(CHUNK, 16)] = jnp.full((16,), -2, jnp.int32)

    zeros_i = jnp.zeros((16,), jnp.int32)
    zeros_f = jnp.zeros((16,), jnp.float32)
    ones_i = jnp.full((16,), 1, jnp.int32)

    def init_body(i, _):
        mark_v[pl.ds(i * 16, 16)] = zeros_i
        vplane_v[pl.ds(i * 16, 16)] = zeros_f
        return 0

    lax.fori_loop(0, CELLS // 16, init_body, 0)
    upd_copy.wait()

    def scan_body(v, _):
        cur = key_v[pl.ds(v * 16, 16)]
        nxt = plsc.load_gather(key_v, [v * 16 + 1 + iota])
        keep = cur != nxt
        val = val_v[pl.ds(v * 16, 16)]
        plsc.store_scatter(mark_v, [cur], ones_i, mask=keep)
        plsc.store_scatter(vplane_v, [cur], val, mask=keep)
        return 0

    lax.fori_loop(0, CHUNK // 16, scan_body, 0)

    pltpu.sync_copy(mark_v, mark_hbm.at[wid])
    pltpu.sync_copy(vplane_v, val_hbm.at[wid])


@functools.cache
def _sc_scatter():
    return pl.kernel(
        _sc_body,
        mesh=plsc.VectorSubcoreMesh(core_axis_name="c", subcore_axis_name="s"),
        out_type=[
            jax.ShapeDtypeStruct((NW, CELLS), jnp.int32),
            jax.ShapeDtypeStruct((NW, CELLS), jnp.float32),
        ],
        scratch_types=[
            pltpu.VMEM((CHUNK + 16,), jnp.int32),
            pltpu.VMEM((CHUNK,), jnp.float32),
            pltpu.VMEM((CELLS,), jnp.int32),
            pltpu.VMEM((CELLS,), jnp.float32),
            pltpu.SemaphoreType.DMA,
        ],
        compiler_params=pltpu.CompilerParams(needs_layout_passes=False),
    )


MR = M // NW
HALF = MR // 2


def _sc_prelude_body(si_hbm, upd_hbm, keys_hbm, updlin_hbm,
                     si_v, upd4_v, key_v, updlin_v, sem_a, sem_b):
    wid = lax.axis_index("s") * 2 + lax.axis_index("c")
    iota = lax.iota(jnp.int32, 16)
    zeros = jnp.zeros((16,), jnp.int32)
    ones = jnp.full((16,), 1, jnp.int32)

    for h in range(2):
        m0 = wid * MR + h * HALF
        ca = pltpu.make_async_copy(si_hbm.at[:, :, pl.ds(m0, HALF)], si_v, sem_a)
        cb = pltpu.make_async_copy(upd_hbm.at[:, pl.ds(m0, HALF)], upd4_v, sem_b)
        ca.start()
        cb.start()
        ca.wait()
        cb.wait()

        def body(v, _):
            jv = v * 16 + iota
            mrel = jv >> 2
            k = jv & 3
            i0 = plsc.load_gather(si_v, [k, zeros, mrel])
            i1 = plsc.load_gather(si_v, [k, ones, mrel])
            key_v[pl.ds(v * 16, 16)] = i0 * C + i1
            updlin_v[pl.ds(v * 16, 16)] = plsc.load_gather(upd4_v, [k, mrel])
            return 0

        lax.fori_loop(0, (HALF * K) // 16, body, 0, unroll=4)
        j0 = wid * (MR * K) + h * (HALF * K)
        pltpu.sync_copy(key_v, keys_hbm.at[pl.ds(j0, HALF * K)])
        pltpu.sync_copy(updlin_v, updlin_hbm.at[pl.ds(j0, HALF * K)])


@functools.cache
def _sc_prelude():
    return pl.kernel(
        _sc_prelude_body,
        mesh=plsc.VectorSubcoreMesh(core_axis_name="c", subcore_axis_name="s"),
        out_type=[
            jax.ShapeDtypeStruct((NU,), jnp.int32),
            jax.ShapeDtypeStruct((NU,), jnp.float32),
        ],
        scratch_types=[
            pltpu.VMEM((K, 2, HALF), jnp.int32),
            pltpu.VMEM((K, HALF), jnp.float32),
            pltpu.VMEM((HALF * K,), jnp.int32),
            pltpu.VMEM((HALF * K,), jnp.float32),
            pltpu.SemaphoreType.DMA,
            pltpu.SemaphoreType.DMA,
        ],
        compiler_params=pltpu.CompilerParams(needs_layout_passes=False),
    )


ROWS_PER_BLK = 4096


def _tc_copy_body(op_ref, out_ref):
    out_ref[...] = op_ref[...]


def _merge_body(buf_ref, mark_ref, val_ref, out_ref):
    patch = buf_ref[...]
    for t in range(NW):
        patch = jnp.where(mark_ref[t] != 0, val_ref[t], patch)
    out_ref[...] = patch


def kernel(operand, scatter_indices, updates):
    si_t = jnp.transpose(scatter_indices.astype(jnp.int32), (1, 2, 0))
    upd_t = jnp.transpose(updates, (1, 0))
    keys, upd = _sc_prelude()(si_t, upd_t)

    out_buf = pl.pallas_call(
        _tc_copy_body,
        grid=(M // ROWS_PER_BLK,),
        in_specs=[pl.BlockSpec((ROWS_PER_BLK, D), lambda i: (i, 0))],
        out_specs=pl.BlockSpec((ROWS_PER_BLK, D), lambda i: (i, 0)),
        out_shape=jax.ShapeDtypeStruct((M, D), jnp.float32),
    )(operand)

    skey, sval = lax.sort((keys, upd), dimension=0, is_stable=False, num_keys=1)
    mark, vals = _sc_scatter()(skey, sval)

    return pl.pallas_call(
        _merge_body,
        grid=(1,),
        in_specs=[
            pl.BlockSpec((R, C), lambda i: (0, 0)),
            pl.BlockSpec((NW, R, C), lambda i: (0, 0, 0)),
            pl.BlockSpec((NW, R, C), lambda i: (0, 0, 0)),
        ],
        out_specs=pl.BlockSpec((R, C), lambda i: (0, 0)),
        out_shape=jax.ShapeDtypeStruct((M, D), jnp.float32),
        input_output_aliases={0: 0},
    )(out_buf, mark.reshape(NW, R, C), vals.reshape(NW, R, C))


# speedup vs baseline: 5.9362x; 1.0082x over previous
import functools

import jax
import jax.numpy as jnp
from jax import lax
from jax.experimental import pallas as pl
from jax.experimental.pallas import tpu as pltpu
from jax.experimental.pallas import tpu_sc as plsc

M = 262144
D = 128
K = 4
NU = M * K
NW = 32
CHUNK = NU // NW
R = 128
C = 128
CELLS = R * C


def _sc_body(skey_hbm, sval_hbm, mark_hbm, val_hbm, key_v, val_v, mark_v, vplane_v, usem):
    wid = lax.axis_index("s") * 2 + lax.axis_index("c")
    base = wid * CHUNK
    upd_copy = pltpu.make_async_copy(sval_hbm.at[pl.ds(base, CHUNK)], val_v, usem)
    upd_copy.start()
    pltpu.sync_copy(skey_hbm.at[pl.ds(base, CHUNK)], key_v.at[pl.ds(0, CHUNK)])

    iota = lax.iota(jnp.int32, 16)

    @pl.when(wid < NW - 1)
    def _():
        pltpu.sync_copy(skey_hbm.at[pl.ds(base + CHUNK, 8)],
                        key_v.at[pl.ds(CHUNK, 8)])

    @pl.when(wid == NW - 1)
    def _():
        key_v[pl.ds(CHUNK, 16)] = jnp.full((16,), -2, jnp.int32)

    zeros_i = jnp.zeros((16,), jnp.int32)
    zeros_f = jnp.zeros((16,), jnp.float32)
    ones_i = jnp.full((16,), 1, jnp.int32)

    def init_body(i, _):
        mark_v[pl.ds(i * 16, 16)] = zeros_i
        vplane_v[pl.ds(i * 16, 16)] = zeros_f
        return 0

    lax.fori_loop(0, CELLS // 16, init_body, 0)
    upd_copy.wait()

    def scan_body(v, _):
        cur = key_v[pl.ds(v * 16, 16)]
        nxt = plsc.load_gather(key_v, [v * 16 + 1 + iota])
        keep = cur != nxt
        val = val_v[pl.ds(v * 16, 16)]
        plsc.store_scatter(mark_v, [cur], ones_i, mask=keep)
        plsc.store_scatter(vplane_v, [cur], val, mask=keep)
        return 0

    lax.fori_loop(0, CHUNK // 16, scan_body, 0)

    pltpu.sync_copy(mark_v, mark_hbm.at[wid])
    pltpu.sync_copy(vplane_v, val_hbm.at[wid])


@functools.cache
def _sc_scatter():
    return pl.kernel(
        _sc_body,
        mesh=plsc.VectorSubcoreMesh(core_axis_name="c", subcore_axis_name="s"),
        out_type=[
            jax.ShapeDtypeStruct((NW, CELLS), jnp.int32),
            jax.ShapeDtypeStruct((NW, CELLS), jnp.float32),
        ],
        scratch_types=[
            pltpu.VMEM((CHUNK + 16,), jnp.int32),
            pltpu.VMEM((CHUNK,), jnp.float32),
            pltpu.VMEM((CELLS,), jnp.int32),
            pltpu.VMEM((CELLS,), jnp.float32),
            pltpu.SemaphoreType.DMA,
        ],
        compiler_params=pltpu.CompilerParams(needs_layout_passes=False),
    )


MR = M // NW
HALF = MR // 2


def _sc_prelude_body(si_hbm, upd_hbm, keys_hbm, updlin_hbm,
                     si_v, upd4_v, key_v, updlin_v, sem_a, sem_b):
    wid = lax.axis_index("s") * 2 + lax.axis_index("c")
    iota = lax.iota(jnp.int32, 16)
    zeros = jnp.zeros((16,), jnp.int32)
    ones = jnp.full((16,), 1, jnp.int32)

    for h in range(2):
        m0 = wid * MR + h * HALF
        ca = pltpu.make_async_copy(si_hbm.at[:, :, pl.ds(m0, HALF)], si_v, sem_a)
        cb = pltpu.make_async_copy(upd_hbm.at[:, pl.ds(m0, HALF)], upd4_v, sem_b)
        ca.start()
        cb.start()
        ca.wait()
        cb.wait()

        def body(v, _):
            jv = v * 16 + iota
            mrel = jv >> 2
            k = jv & 3
            i0 = plsc.load_gather(si_v, [k, zeros, mrel])
            i1 = plsc.load_gather(si_v, [k, ones, mrel])
            key_v[pl.ds(v * 16, 16)] = i0 * C + i1
            updlin_v[pl.ds(v * 16, 16)] = plsc.load_gather(upd4_v, [k, mrel])
            return 0

        lax.fori_loop(0, (HALF * K) // 16, body, 0, unroll=4)
        j0 = wid * (MR * K) + h * (HALF * K)
        pltpu.sync_copy(key_v, keys_hbm.at[pl.ds(j0, HALF * K)])
        pltpu.sync_copy(updlin_v, updlin_hbm.at[pl.ds(j0, HALF * K)])


@functools.cache
def _sc_prelude():
    return pl.kernel(
        _sc_prelude_body,
        mesh=plsc.VectorSubcoreMesh(core_axis_name="c", subcore_axis_name="s"),
        out_type=[
            jax.ShapeDtypeStruct((NU,), jnp.int32),
            jax.ShapeDtypeStruct((NU,), jnp.float32),
        ],
        scratch_types=[
            pltpu.VMEM((K, 2, HALF), jnp.int32),
            pltpu.VMEM((K, HALF), jnp.float32),
            pltpu.VMEM((HALF * K,), jnp.int32),
            pltpu.VMEM((HALF * K,), jnp.float32),
            pltpu.SemaphoreType.DMA,
            pltpu.SemaphoreType.DMA,
        ],
        compiler_params=pltpu.CompilerParams(needs_layout_passes=False),
    )


ROWS_PER_BLK = 8192


def _tc_copy_body(op_ref, out_ref):
    out_ref[...] = op_ref[...]


def _merge_body(buf_ref, mark_ref, val_ref, out_ref):
    patch = buf_ref[...]
    for t in range(NW):
        patch = jnp.where(mark_ref[t] != 0, val_ref[t], patch)
    out_ref[...] = patch


def kernel(operand, scatter_indices, updates):
    si_t = jnp.transpose(scatter_indices.astype(jnp.int32), (1, 2, 0))
    upd_t = jnp.transpose(updates, (1, 0))
    keys, upd = _sc_prelude()(si_t, upd_t)

    out_buf = pl.pallas_call(
        _tc_copy_body,
        grid=(M // ROWS_PER_BLK,),
        in_specs=[pl.BlockSpec((ROWS_PER_BLK, D), lambda i: (i, 0))],
        out_specs=pl.BlockSpec((ROWS_PER_BLK, D), lambda i: (i, 0)),
        out_shape=jax.ShapeDtypeStruct((M, D), jnp.float32),
    )(operand)

    skey, sval = lax.sort((keys, upd), dimension=0, is_stable=False, num_keys=1)
    mark, vals = _sc_scatter()(skey, sval)

    return pl.pallas_call(
        _merge_body,
        grid=(1,),
        in_specs=[
            pl.BlockSpec((R, C), lambda i: (0, 0)),
            pl.BlockSpec((NW, R, C), lambda i: (0, 0, 0)),
            pl.BlockSpec((NW, R, C), lambda i: (0, 0, 0)),
        ],
        out_specs=pl.BlockSpec((R, C), lambda i: (0, 0)),
        out_shape=jax.ShapeDtypeStruct((M, D), jnp.float32),
        input_output_aliases={0: 0},
    )(out_buf, mark.reshape(NW, R, C), vals.reshape(NW, R, C))


# speedup vs baseline: 5.9457x; 1.0016x over previous
import functools

import jax
import jax.numpy as jnp
from jax import lax
from jax.experimental import pallas as pl
from jax.experimental.pallas import tpu as pltpu
from jax.experimental.pallas import tpu_sc as plsc

M = 262144
D = 128
K = 4
NU = M * K
NW = 32
CHUNK = NU // NW
R = 128
C = 128
CELLS = R * C


def _sc_body(skey_hbm, sval_hbm, mark_hbm, val_hbm, key_v, val_v, mark_v, vplane_v, usem):
    wid = lax.axis_index("s") * 2 + lax.axis_index("c")
    base = wid * CHUNK
    upd_copy = pltpu.make_async_copy(sval_hbm.at[pl.ds(base, CHUNK)], val_v, usem)
    upd_copy.start()
    pltpu.sync_copy(skey_hbm.at[pl.ds(base, CHUNK)], key_v.at[pl.ds(0, CHUNK)])

    iota = lax.iota(jnp.int32, 16)

    @pl.when(wid < NW - 1)
    def _():
        pltpu.sync_copy(skey_hbm.at[pl.ds(base + CHUNK, 8)],
                        key_v.at[pl.ds(CHUNK, 8)])

    @pl.when(wid == NW - 1)
    def _():
        key_v[pl.ds(CHUNK, 16)] = jnp.full((16,), -2, jnp.int32)

    zeros_i = jnp.zeros((16,), jnp.int32)
    zeros_f = jnp.zeros((16,), jnp.float32)
    ones_i = jnp.full((16,), 1, jnp.int32)

    def init_body(i, _):
        mark_v[pl.ds(i * 16, 16)] = zeros_i
        vplane_v[pl.ds(i * 16, 16)] = zeros_f
        return 0

    lax.fori_loop(0, CELLS // 16, init_body, 0)
    upd_copy.wait()

    def scan_body(v, _):
        cur = key_v[pl.ds(v * 16, 16)]
        nxt = plsc.load_gather(key_v, [v * 16 + 1 + iota])
        keep = cur != nxt
        val = val_v[pl.ds(v * 16, 16)]
        plsc.store_scatter(mark_v, [cur], ones_i, mask=keep)
        plsc.store_scatter(vplane_v, [cur], val, mask=keep)
        return 0

    lax.fori_loop(0, CHUNK // 16, scan_body, 0)

    pltpu.sync_copy(mark_v, mark_hbm.at[wid])
    pltpu.sync_copy(vplane_v, val_hbm.at[wid])


@functools.cache
def _sc_scatter():
    return pl.kernel(
        _sc_body,
        mesh=plsc.VectorSubcoreMesh(core_axis_name="c", subcore_axis_name="s"),
        out_type=[
            jax.ShapeDtypeStruct((NW, CELLS), jnp.int32),
            jax.ShapeDtypeStruct((NW, CELLS), jnp.float32),
        ],
        scratch_types=[
            pltpu.VMEM((CHUNK + 16,), jnp.int32),
            pltpu.VMEM((CHUNK,), jnp.float32),
            pltpu.VMEM((CELLS,), jnp.int32),
            pltpu.VMEM((CELLS,), jnp.float32),
            pltpu.SemaphoreType.DMA,
        ],
        compiler_params=pltpu.CompilerParams(needs_layout_passes=False),
    )


MR = M // NW
HALF = MR // 2


def _sc_prelude_body(si_hbm, upd_hbm, keys_hbm, updlin_hbm,
                     si_v, upd4_v, key_v, updlin_v, sem_a, sem_b):
    wid = lax.axis_index("s") * 2 + lax.axis_index("c")
    iota = lax.iota(jnp.int32, 16)
    zeros = jnp.zeros((16,), jnp.int32)
    ones = jnp.full((16,), 1, jnp.int32)

    for h in range(2):
        m0 = wid * MR + h * HALF
        ca = pltpu.make_async_copy(si_hbm.at[:, :, pl.ds(m0, HALF)], si_v, sem_a)
        cb = pltpu.make_async_copy(upd_hbm.at[:, pl.ds(m0, HALF)], upd4_v, sem_b)
        ca.start()
        cb.start()
        ca.wait()
        cb.wait()

        def body(v, _):
            jv = v * 16 + iota
            mrel = jv >> 2
            k = jv & 3
            i0 = plsc.load_gather(si_v, [k, zeros, mrel])
            i1 = plsc.load_gather(si_v, [k, ones, mrel])
            key_v[pl.ds(v * 16, 16)] = i0 * C + i1
            updlin_v[pl.ds(v * 16, 16)] = plsc.load_gather(upd4_v, [k, mrel])
            return 0

        lax.fori_loop(0, (HALF * K) // 16, body, 0, unroll=4)
        j0 = wid * (MR * K) + h * (HALF * K)
        pltpu.sync_copy(key_v, keys_hbm.at[pl.ds(j0, HALF * K)])
        pltpu.sync_copy(updlin_v, updlin_hbm.at[pl.ds(j0, HALF * K)])


@functools.cache
def _sc_prelude():
    return pl.kernel(
        _sc_prelude_body,
        mesh=plsc.VectorSubcoreMesh(core_axis_name="c", subcore_axis_name="s"),
        out_type=[
            jax.ShapeDtypeStruct((NU,), jnp.int32),
            jax.ShapeDtypeStruct((NU,), jnp.float32),
        ],
        scratch_types=[
            pltpu.VMEM((K, 2, HALF), jnp.int32),
            pltpu.VMEM((K, HALF), jnp.float32),
            pltpu.VMEM((HALF * K,), jnp.int32),
            pltpu.VMEM((HALF * K,), jnp.float32),
            pltpu.SemaphoreType.DMA,
            pltpu.SemaphoreType.DMA,
        ],
        compiler_params=pltpu.CompilerParams(needs_layout_passes=False),
    )


ROWS_PER_BLK = 16384


def _tc_copy_body(op_ref, out_ref):
    out_ref[...] = op_ref[...]


def _merge_body(buf_ref, mark_ref, val_ref, out_ref):
    patch = buf_ref[...]
    for t in range(NW):
        patch = jnp.where(mark_ref[t] != 0, val_ref[t], patch)
    out_ref[...] = patch


def kernel(operand, scatter_indices, updates):
    si_t = jnp.transpose(scatter_indices.astype(jnp.int32), (1, 2, 0))
    upd_t = jnp.transpose(updates, (1, 0))
    keys, upd = _sc_prelude()(si_t, upd_t)

    out_buf = pl.pallas_call(
        _tc_copy_body,
        grid=(M // ROWS_PER_BLK,),
        in_specs=[pl.BlockSpec((ROWS_PER_BLK, D), lambda i: (i, 0))],
        out_specs=pl.BlockSpec((ROWS_PER_BLK, D), lambda i: (i, 0)),
        out_shape=jax.ShapeDtypeStruct((M, D), jnp.float32),
    )(operand)

    skey, sval = lax.sort((keys, upd), dimension=0, is_stable=False, num_keys=1)
    mark, vals = _sc_scatter()(skey, sval)

    return pl.pallas_call(
        _merge_body,
        grid=(1,),
        in_specs=[
            pl.BlockSpec((R, C), lambda i: (0, 0)),
            pl.BlockSpec((NW, R, C), lambda i: (0, 0, 0)),
            pl.BlockSpec((NW, R, C), lambda i: (0, 0, 0)),
        ],
        out_specs=pl.BlockSpec((R, C), lambda i: (0, 0)),
        out_shape=jax.ShapeDtypeStruct((M, D), jnp.float32),
        input_output_aliases={0: 0},
    )(out_buf, mark.reshape(NW, R, C), vals.reshape(NW, R, C))
